# baseline (speedup 1.0000x reference)
import functools

import jax
import jax.numpy as jnp
from jax import lax
from jax.experimental import pallas as pl
from jax.experimental.pallas import tpu as pltpu
from jax.experimental.pallas import tpu_sc as plsc

V = 1000000
D = 64
H = 128
A = 6
B = 4096
T = 50

NC = 2
NS = 16
NW = NC * NS
ROWS_TOTAL = B * T
ROWS_PER_W = ROWS_TOTAL // NW
CHUNK = 128
NCHUNK = ROWS_PER_W // CHUNK


@functools.lru_cache(maxsize=1)
def _make_sc_gather():
    mesh = plsc.VectorSubcoreMesh(core_axis_name="c", subcore_axis_name="s")

    @functools.partial(
        pl.kernel,
        mesh=mesh,
        out_type=jax.ShapeDtypeStruct((ROWS_TOTAL, 2 * D), jnp.float32),
        scratch_types=[
            pltpu.VMEM((ROWS_PER_W,), jnp.int32),
            pltpu.VMEM((CHUNK, 2 * D), jnp.float32),
            pltpu.SemaphoreType.DMA,
        ],
    )
    def gather_k(emb_hbm, idx_hbm, out_hbm, idx_v, buf, gsem):
        wid = lax.axis_index("s") * NC + lax.axis_index("c")
        base = wid * ROWS_PER_W
        pltpu.sync_copy(idx_hbm.at[pl.ds(base, ROWS_PER_W)], idx_v)

        def chunk_body(j, carry):
            pltpu.async_copy(
                emb_hbm.at[idx_v.at[pl.ds(j * CHUNK, CHUNK)]], buf, gsem
            ).wait()
            pltpu.sync_copy(buf, out_hbm.at[pl.ds(base + j * CHUNK, CHUNK)])
            return carry

        lax.fori_loop(0, NCHUNK, chunk_body, 0)

    return gather_k


NCOL = 4096
HALF = 512000
NBLK = HALF // NCOL


def _trans_body(in0_ref, in1_ref, out_ref):
    out_ref[:, :D] = jnp.swapaxes(in0_ref[...], 0, 1)
    out_ref[:, D:] = jnp.swapaxes(in1_ref[...], 0, 1)


def _pack_table(embT):
    return pl.pallas_call(
        _trans_body,
        grid=(NBLK,),
        in_specs=[
            pl.BlockSpec((D, NCOL), lambda i: (0, i)),
            pl.BlockSpec((D, NCOL), lambda i: (0, jnp.minimum(i + NBLK, V // NCOL))),
        ],
        out_specs=pl.BlockSpec((NCOL, 2 * D), lambda i: (i, 0)),
        out_shape=jax.ShapeDtypeStruct((HALF, 2 * D), jnp.float32),
    )(embT, embT)


BB = 512


def _sigmoid(x):
    return 0.5 * jnp.tanh(0.5 * x) + 0.5


def _lstm_body(x_ref, p_ref, w_ref, b_ref, wfc_ref, bfc_ref, out_ref):
    w = w_ref[...]
    b = b_ref[...]

    h = jnp.zeros((BB, H), jnp.float32)
    c = jnp.zeros((BB, H), jnp.float32)
    for t in range(T):
        x2 = x_ref[t]
        sb = p_ref[:, t : t + 1] != 0
        xt = jnp.where(sb, x2[:, D:], x2[:, :D]).astype(jnp.bfloat16)
        xh = jnp.concatenate([xt, h.astype(jnp.bfloat16)], axis=1)
        gates = jnp.dot(xh, w, preferred_element_type=jnp.float32) + b
        i = _sigmoid(gates[:, 0:H])
        f = _sigmoid(gates[:, H : 2 * H])
        g = jnp.tanh(gates[:, 2 * H : 3 * H])
        o = _sigmoid(gates[:, 3 * H : 4 * H])
        c = f * c + i * g
        h = o * jnp.tanh(c)
    logits = jnp.dot(h, wfc_ref[...], preferred_element_type=jnp.float32) + bfc_ref[...]
    m = jnp.max(logits, axis=-1, keepdims=True)
    lse = jnp.log(jnp.sum(jnp.exp(logits - m), axis=-1, keepdims=True)) + m
    out_ref[...] = logits - lse


def _lstm_call(x, par, w_cat, bias, wfc_pad, bfc_pad):
    return pl.pallas_call(
        _lstm_body,
        grid=(B // BB,),
        in_specs=[
            pl.BlockSpec((T, BB, 2 * D), lambda i: (0, i, 0)),
            pl.BlockSpec((BB, T), lambda i: (i, 0)),
            pl.BlockSpec((D + H, 4 * H), lambda i: (0, 0)),
            pl.BlockSpec((1, 4 * H), lambda i: (0, 0)),
            pl.BlockSpec((H, 128), lambda i: (0, 0)),
            pl.BlockSpec((1, 128), lambda i: (0, 0)),
        ],
        out_specs=pl.BlockSpec((BB, 128), lambda i: (i, 0)),
        out_shape=jax.ShapeDtypeStruct((B, 128), jnp.float32),
    )(x, par, w_cat, bias, wfc_pad, bfc_pad)


def kernel(dct_in, emb, W_ih, W_hh, b_ih, b_hh, W_fc, b_fc):
    dct_t = jnp.swapaxes(dct_in, 0, 1).astype(jnp.int32)
    hi = dct_t >= HALF
    idx = jnp.where(hi, dct_t - HALF, dct_t).reshape(ROWS_TOTAL)
    parity = (dct_in >= HALF).astype(jnp.int32)

    emb_w = _pack_table(jnp.swapaxes(emb, 0, 1))
    x2_flat = _make_sc_gather()(emb_w, idx)
    x2 = x2_flat.reshape(T, B, 2 * D)

    w_cat = jnp.concatenate([W_ih.T, W_hh.T], axis=0).astype(jnp.bfloat16)
    bias = (b_ih + b_hh).reshape(1, 4 * H)
    wfc_pad = jnp.zeros((H, 128), jnp.float32).at[:, :A].set(W_fc.T)
    bfc_pad = jnp.full((1, 128), -1e30, jnp.float32).at[0, :A].set(b_fc)

    out = _lstm_call(x2, parity, w_cat, bias, wfc_pad, bfc_pad)
    return out[:, :A]


# speedup vs baseline: 1.0015x; 1.0015x over previous
import functools

import jax
import jax.numpy as jnp
from jax import lax
from jax.experimental import pallas as pl
from jax.experimental.pallas import tpu as pltpu
from jax.experimental.pallas import tpu_sc as plsc

V = 1000000
D = 64
H = 128
A = 6
B = 4096
T = 50

NC = 2
NS = 16
NW = NC * NS
ROWS_TOTAL = B * T
ROWS_PER_W = ROWS_TOTAL // NW
CHUNK = 128
NCHUNK = ROWS_PER_W // CHUNK


@functools.lru_cache(maxsize=1)
def _make_sc_gather():
    mesh = plsc.VectorSubcoreMesh(core_axis_name="c", subcore_axis_name="s")

    @functools.partial(
        pl.kernel,
        mesh=mesh,
        out_type=jax.ShapeDtypeStruct((ROWS_TOTAL, 2 * D), jnp.float32),
        scratch_types=[
            pltpu.VMEM((ROWS_PER_W,), jnp.int32),
            pltpu.VMEM((CHUNK, 2 * D), jnp.float32),
            pltpu.SemaphoreType.DMA,
        ],
    )
    def gather_k(emb_hbm, idx_hbm, out_hbm, idx_v, buf, gsem):
        wid = lax.axis_index("s") * NC + lax.axis_index("c")
        base = wid * ROWS_PER_W
        pltpu.sync_copy(idx_hbm.at[pl.ds(base, ROWS_PER_W)], idx_v)

        def chunk_body(j, carry):
            pltpu.async_copy(
                emb_hbm.at[idx_v.at[pl.ds(j * CHUNK, CHUNK)]], buf, gsem
            ).wait()
            pltpu.sync_copy(buf, out_hbm.at[pl.ds(base + j * CHUNK, CHUNK)])
            return carry

        lax.fori_loop(0, NCHUNK, chunk_body, 0)

    return gather_k


NCOL = 4096
HALF = 512000
NBLK = HALF // NCOL


def _trans_body(in0_ref, in1_ref, out_ref):
    eye = jnp.eye(D, dtype=jnp.float32)
    dn = (((0,), (0,)), ((), ()))
    out_ref[:, :D] = lax.dot_general(
        in0_ref[...], eye, dn, preferred_element_type=jnp.float32
    )
    out_ref[:, D:] = lax.dot_general(
        in1_ref[...], eye, dn, preferred_element_type=jnp.float32
    )


def _pack_table(embT):
    return pl.pallas_call(
        _trans_body,
        grid=(NBLK,),
        in_specs=[
            pl.BlockSpec((D, NCOL), lambda i: (0, i)),
            pl.BlockSpec((D, NCOL), lambda i: (0, jnp.minimum(i + NBLK, V // NCOL))),
        ],
        out_specs=pl.BlockSpec((NCOL, 2 * D), lambda i: (i, 0)),
        out_shape=jax.ShapeDtypeStruct((HALF, 2 * D), jnp.float32),
    )(embT, embT)


BB = 512


def _sigmoid(x):
    return 0.5 * jnp.tanh(0.5 * x) + 0.5


def _lstm_body(x_ref, p_ref, w_ref, b_ref, wfc_ref, bfc_ref, out_ref):
    w = w_ref[...]
    b = b_ref[...]

    h = jnp.zeros((BB, H), jnp.float32)
    c = jnp.zeros((BB, H), jnp.float32)
    for t in range(T):
        x2 = x_ref[t]
        sb = p_ref[:, t : t + 1] != 0
        xt = jnp.where(sb, x2[:, D:], x2[:, :D]).astype(jnp.bfloat16)
        xh = jnp.concatenate([xt, h.astype(jnp.bfloat16)], axis=1)
        gates = jnp.dot(xh, w, preferred_element_type=jnp.float32) + b
        i = _sigmoid(gates[:, 0:H])
        f = _sigmoid(gates[:, H : 2 * H])
        g = jnp.tanh(gates[:, 2 * H : 3 * H])
        o = _sigmoid(gates[:, 3 * H : 4 * H])
        c = f * c + i * g
        h = o * jnp.tanh(c)
    logits = jnp.dot(h, wfc_ref[...], preferred_element_type=jnp.float32) + bfc_ref[...]
    m = jnp.max(logits, axis=-1, keepdims=True)
    lse = jnp.log(jnp.sum(jnp.exp(logits - m), axis=-1, keepdims=True)) + m
    out_ref[...] = logits - lse


def _lstm_call(x, par, w_cat, bias, wfc_pad, bfc_pad):
    return pl.pallas_call(
        _lstm_body,
        grid=(B // BB,),
        in_specs=[
            pl.BlockSpec((T, BB, 2 * D), lambda i: (0, i, 0)),
            pl.BlockSpec((BB, T), lambda i: (i, 0)),
            pl.BlockSpec((D + H, 4 * H), lambda i: (0, 0)),
            pl.BlockSpec((1, 4 * H), lambda i: (0, 0)),
            pl.BlockSpec((H, 128), lambda i: (0, 0)),
            pl.BlockSpec((1, 128), lambda i: (0, 0)),
        ],
        out_specs=pl.BlockSpec((BB, 128), lambda i: (i, 0)),
        out_shape=jax.ShapeDtypeStruct((B, 128), jnp.float32),
    )(x, par, w_cat, bias, wfc_pad, bfc_pad)


def kernel(dct_in, emb, W_ih, W_hh, b_ih, b_hh, W_fc, b_fc):
    dct_t = jnp.swapaxes(dct_in, 0, 1).astype(jnp.int32)
    hi = dct_t >= HALF
    idx = jnp.where(hi, dct_t - HALF, dct_t).reshape(ROWS_TOTAL)
    parity = (dct_in >= HALF).astype(jnp.int32)

    emb_w = _pack_table(jnp.swapaxes(emb, 0, 1))
    x2_flat = _make_sc_gather()(emb_w, idx)
    x2 = x2_flat.reshape(T, B, 2 * D)

    w_cat = jnp.concatenate([W_ih.T, W_hh.T], axis=0).astype(jnp.bfloat16)
    bias = (b_ih + b_hh).reshape(1, 4 * H)
    wfc_pad = jnp.zeros((H, 128), jnp.float32).at[:, :A].set(W_fc.T)
    bfc_pad = jnp.full((1, 128), -1e30, jnp.float32).at[0, :A].set(b_fc)

    out = _lstm_call(x2, parity, w_cat, bias, wfc_pad, bfc_pad)
    return out[:, :A]


# speedup vs baseline: 1.1424x; 1.1407x over previous
import functools

import jax
import jax.numpy as jnp
from jax import lax
from jax.experimental import pallas as pl
from jax.experimental.pallas import tpu as pltpu
from jax.experimental.pallas import tpu_sc as plsc

V = 1000000
D = 64
H = 128
A = 6
B = 4096
T = 50

NC = 2
NS = 16
NW = NC * NS
ROWS_TOTAL = B * T
ROWS_PER_W = ROWS_TOTAL // NW
CHUNK = 128
NCHUNK = ROWS_PER_W // CHUNK


@functools.lru_cache(maxsize=1)
def _make_sc_gather():
    mesh = plsc.VectorSubcoreMesh(core_axis_name="c", subcore_axis_name="s")

    @functools.partial(
        pl.kernel,
        mesh=mesh,
        out_type=jax.ShapeDtypeStruct((ROWS_TOTAL, 2 * D), jnp.float32),
        scratch_types=[
            pltpu.VMEM((ROWS_PER_W,), jnp.int32),
            pltpu.VMEM((CHUNK, 2 * D), jnp.float32),
            pltpu.SemaphoreType.DMA,
        ],
    )
    def gather_k(emb_hbm, idx_hbm, out_hbm, idx_v, buf, gsem):
        wid = lax.axis_index("s") * NC + lax.axis_index("c")
        base = wid * ROWS_PER_W
        pltpu.sync_copy(idx_hbm.at[pl.ds(base, ROWS_PER_W)], idx_v)

        def chunk_body(j, carry):
            pltpu.async_copy(
                emb_hbm.at[idx_v.at[pl.ds(j * CHUNK, CHUNK)]], buf, gsem
            ).wait()
            pltpu.sync_copy(buf, out_hbm.at[pl.ds(base + j * CHUNK, CHUNK)])
            return carry

        lax.fori_loop(0, NCHUNK, chunk_body, 0)

    return gather_k


NCOL = 16384


def _trans_body(in_ref, out_ref):
    a = in_ref[...]
    out_ref[:, :D] = jnp.swapaxes(a, 0, 1)


def _widen_table(embT):
    return pl.pallas_call(
        _trans_body,
        grid=((V + NCOL - 1) // NCOL,),
        in_specs=[pl.BlockSpec((D, NCOL), lambda i: (0, i))],
        out_specs=pl.BlockSpec((NCOL, 2 * D), lambda i: (i, 0)),
        out_shape=jax.ShapeDtypeStruct((V, 2 * D), jnp.float32),
    )(embT)


BB = 1024


def _lstm_body(x_ref, w_ref, b_ref, wfc_ref, bfc_ref, out_ref):
    w = w_ref[...]
    b = b_ref[...]

    h = jnp.zeros((BB, H), jnp.float32)
    c = jnp.zeros((BB, H), jnp.float32)
    for t in range(T):
        xt = x_ref[t][:, :D].astype(jnp.bfloat16)
        xh = jnp.concatenate([xt, h.astype(jnp.bfloat16)], axis=1)
        gates = jnp.dot(xh, w, preferred_element_type=jnp.float32) + b
        i = 0.5 * jnp.tanh(gates[:, 0:H]) + 0.5
        f = 0.5 * jnp.tanh(gates[:, H : 2 * H]) + 0.5
        g = jnp.tanh(gates[:, 2 * H : 3 * H])
        o = 0.5 * jnp.tanh(gates[:, 3 * H : 4 * H]) + 0.5
        c = f * c + i * g
        h = o * jnp.tanh(c)
    logits = jnp.dot(h, wfc_ref[...], preferred_element_type=jnp.float32) + bfc_ref[...]
    m = jnp.max(logits, axis=-1, keepdims=True)
    lse = jnp.log(jnp.sum(jnp.exp(logits - m), axis=-1, keepdims=True)) + m
    out_ref[...] = logits - lse


def _lstm_call(x, w_cat, bias, wfc_pad, bfc_pad):
    return pl.pallas_call(
        _lstm_body,
        grid=(B // BB,),
        in_specs=[
            pl.BlockSpec((T, BB, 2 * D), lambda i: (0, i, 0)),
            pl.BlockSpec((D + H, 4 * H), lambda i: (0, 0)),
            pl.BlockSpec((1, 4 * H), lambda i: (0, 0)),
            pl.BlockSpec((H, 128), lambda i: (0, 0)),
            pl.BlockSpec((1, 128), lambda i: (0, 0)),
        ],
        out_specs=pl.BlockSpec((BB, 128), lambda i: (i, 0)),
        out_shape=jax.ShapeDtypeStruct((B, 128), jnp.float32),
    )(x, w_cat, bias, wfc_pad, bfc_pad)


_GATE_SCALE = jnp.concatenate(
    [jnp.full((H,), 0.5), jnp.full((H,), 0.5), jnp.ones((H,)), jnp.full((H,), 0.5)]
)


def kernel(dct_in, emb, W_ih, W_hh, b_ih, b_hh, W_fc, b_fc):
    idx = jnp.swapaxes(dct_in, 0, 1).reshape(ROWS_TOTAL).astype(jnp.int32)

    emb_w = _widen_table(jnp.swapaxes(emb, 0, 1))
    x2_flat = _make_sc_gather()(emb_w, idx)
    x2 = x2_flat.reshape(T, B, 2 * D)

    w_cat = (
        jnp.concatenate([W_ih.T, W_hh.T], axis=0) * _GATE_SCALE
    ).astype(jnp.bfloat16)
    bias = ((b_ih + b_hh) * _GATE_SCALE).reshape(1, 4 * H)
    wfc_pad = jnp.zeros((H, 128), jnp.float32).at[:, :A].set(W_fc.T)
    bfc_pad = jnp.full((1, 128), -1e30, jnp.float32).at[0, :A].set(b_fc)

    out = _lstm_call(x2, w_cat, bias, wfc_pad, bfc_pad)
    return out[:, :A]


# speedup vs baseline: 1.1583x; 1.0139x over previous
import functools

import jax
import jax.numpy as jnp
from jax import lax
from jax.experimental import pallas as pl
from jax.experimental.pallas import tpu as pltpu
from jax.experimental.pallas import tpu_sc as plsc

V = 1000000
D = 64
H = 128
A = 6
B = 4096
T = 50

NC = 2
NS = 16
NW = NC * NS
ROWS_TOTAL = B * T
ROWS_PER_W = ROWS_TOTAL // NW
CHUNK = 128
NCHUNK = ROWS_PER_W // CHUNK


@functools.lru_cache(maxsize=1)
def _make_sc_gather():
    mesh = plsc.VectorSubcoreMesh(core_axis_name="c", subcore_axis_name="s")

    @functools.partial(
        pl.kernel,
        mesh=mesh,
        out_type=jax.ShapeDtypeStruct((ROWS_TOTAL, 2 * D), jnp.float32),
        scratch_types=[
            pltpu.VMEM((ROWS_PER_W,), jnp.int32),
            pltpu.VMEM((CHUNK, 2 * D), jnp.float32),
            pltpu.SemaphoreType.DMA,
        ],
    )
    def gather_k(emb_hbm, idx_hbm, out_hbm, idx_v, buf, gsem):
        wid = lax.axis_index("s") * NC + lax.axis_index("c")
        base = wid * ROWS_PER_W
        pltpu.sync_copy(idx_hbm.at[pl.ds(base, ROWS_PER_W)], idx_v)

        def chunk_body(j, carry):
            pltpu.async_copy(
                emb_hbm.at[idx_v.at[pl.ds(j * CHUNK, CHUNK)]], buf, gsem
            ).wait()
            pltpu.sync_copy(buf, out_hbm.at[pl.ds(base + j * CHUNK, CHUNK)])
            return carry

        lax.fori_loop(0, NCHUNK, chunk_body, 0)

    return gather_k


NCOL = 32768


def _trans_body(in_ref, out_ref):
    a = in_ref[...]
    out_ref[:, :D] = jnp.swapaxes(a, 0, 1)


def _widen_table(embT):
    return pl.pallas_call(
        _trans_body,
        grid=((V + NCOL - 1) // NCOL,),
        in_specs=[pl.BlockSpec((D, NCOL), lambda i: (0, i))],
        out_specs=pl.BlockSpec((NCOL, 2 * D), lambda i: (i, 0)),
        out_shape=jax.ShapeDtypeStruct((V, 2 * D), jnp.float32),
    )(embT)


BB = 1024


def _lstm_body(x_ref, w_ref, b_ref, wfc_ref, bfc_ref, out_ref):
    w = w_ref[...]
    b = b_ref[...]

    h = jnp.zeros((BB, H), jnp.float32)
    c = jnp.zeros((BB, H), jnp.float32)
    for t in range(T):
        xt = x_ref[t][:, :D].astype(jnp.bfloat16)
        xh = jnp.concatenate([xt, h.astype(jnp.bfloat16)], axis=1)
        gates = jnp.dot(xh, w, preferred_element_type=jnp.float32) + b
        i = 0.5 * jnp.tanh(gates[:, 0:H]) + 0.5
        f = 0.5 * jnp.tanh(gates[:, H : 2 * H]) + 0.5
        g = jnp.tanh(gates[:, 2 * H : 3 * H])
        o = 0.5 * jnp.tanh(gates[:, 3 * H : 4 * H]) + 0.5
        c = f * c + i * g
        h = o * jnp.tanh(c)
    logits = jnp.dot(h, wfc_ref[...], preferred_element_type=jnp.float32) + bfc_ref[...]
    m = jnp.max(logits, axis=-1, keepdims=True)
    lse = jnp.log(jnp.sum(jnp.exp(logits - m), axis=-1, keepdims=True)) + m
    out_ref[...] = logits - lse


def _lstm_call(x, w_cat, bias, wfc_pad, bfc_pad):
    return pl.pallas_call(
        _lstm_body,
        grid=(B // BB,),
        in_specs=[
            pl.BlockSpec((T, BB, 2 * D), lambda i: (0, i, 0)),
            pl.BlockSpec((D + H, 4 * H), lambda i: (0, 0)),
            pl.BlockSpec((1, 4 * H), lambda i: (0, 0)),
            pl.BlockSpec((H, 128), lambda i: (0, 0)),
            pl.BlockSpec((1, 128), lambda i: (0, 0)),
        ],
        out_specs=pl.BlockSpec((BB, 128), lambda i: (i, 0)),
        out_shape=jax.ShapeDtypeStruct((B, 128), jnp.float32),
    )(x, w_cat, bias, wfc_pad, bfc_pad)


_GATE_SCALE = jnp.concatenate(
    [jnp.full((H,), 0.5), jnp.full((H,), 0.5), jnp.ones((H,)), jnp.full((H,), 0.5)]
)


def kernel(dct_in, emb, W_ih, W_hh, b_ih, b_hh, W_fc, b_fc):
    idx = jnp.swapaxes(dct_in, 0, 1).reshape(ROWS_TOTAL).astype(jnp.int32)

    emb_w = _widen_table(jnp.swapaxes(emb, 0, 1))
    x2_flat = _make_sc_gather()(emb_w, idx)
    x2 = x2_flat.reshape(T, B, 2 * D)

    w_cat = (
        jnp.concatenate([W_ih.T, W_hh.T], axis=0) * _GATE_SCALE
    ).astype(jnp.bfloat16)
    bias = ((b_ih + b_hh) * _GATE_SCALE).reshape(1, 4 * H)
    wfc_pad = jnp.zeros((H, 128), jnp.float32).at[:, :A].set(W_fc.T)
    bfc_pad = jnp.full((1, 128), -1e30, jnp.float32).at[0, :A].set(b_fc)

    out = _lstm_call(x2, w_cat, bias, wfc_pad, bfc_pad)
    return out[:, :A]
